# baseline (speedup 1.0000x reference)
import functools

import jax
import jax.numpy as jnp
from jax import lax
from jax.experimental import pallas as pl
from jax.experimental.pallas import tpu as pltpu
from jax.experimental.pallas import tpu_sc as plsc

VOCAB = 1000000
DIM = 64

NC = 2
NS = 16
NW = NC * NS

NB = 4096
NT = 200
BBLK = NB // NW
NGB = 4
NOB = 2
LANES = 16


def _make_kernel():
  mesh = plsc.VectorSubcoreMesh(core_axis_name="c", subcore_axis_name="s")

  @functools.partial(
      pl.kernel,
      mesh=mesh,
      compiler_params=pltpu.CompilerParams(
          use_tc_tiling_on_sc=False, needs_layout_passes=False),
      out_type=jax.ShapeDtypeStruct((NT, DIM, NB), jnp.float32),
      scratch_types=[
          pltpu.VMEM((NT, BBLK), jnp.int32),
      ]
      + [pltpu.VMEM((BBLK, DIM), jnp.float32) for _ in range(NGB)]
      + [pltpu.VMEM((NOB, DIM, BBLK), jnp.float32),
         pltpu.VMEM((BBLK * DIM,), jnp.float32)]
      + [
          pltpu.SemaphoreType.DMA((NGB,)),
          pltpu.SemaphoreType.DMA((NOB,)),
      ],
  )
  def gather_kernel(idx_hbm, table_hbm, out_hbm, ibuf, g0, g1, g2, g3,
                    obuf, fbuf, gsem, osem):
    gbufs = (g0, g1, g2, g3)
    wid = lax.axis_index("s") * NC + lax.axis_index("c")
    b0 = wid * BBLK

    pltpu.sync_copy(idx_hbm.at[:, pl.ds(b0, BBLK)], ibuf)

    def issue_gather(p, sg):
      pltpu.async_copy(table_hbm.at[ibuf.at[p]], gbufs[sg], gsem.at[sg])

    def wait_gather(sg):
      pltpu.make_async_copy(
          table_hbm.at[ibuf.at[0]], gbufs[sg], gsem.at[sg]).wait()

    def issue_out(p, so):
      pltpu.async_copy(
          obuf.at[so], out_hbm.at[p, :, pl.ds(b0, BBLK)], osem.at[so])

    def wait_out(so):
      pltpu.make_async_copy(
          obuf.at[so], out_hbm.at[0, :, pl.ds(b0, BBLK)], osem.at[so]).wait()

    def transpose(sg, so):
      def jloop(j4, _):
        for jj in range(4):
          j = j4 * 4 + jj
          for dg in range(DIM // LANES):
            fbuf[pl.ds(j * DIM + dg * LANES, LANES)] = (
                gbufs[sg][j, pl.ds(dg * LANES, LANES)])
        return _

      lax.fori_loop(0, BBLK // 4, jloop, 0, unroll=False)

      def dloop(d8, _):
        for dd in range(8):
          d = d8 * 8 + dd
          for jg in range(BBLK // LANES):
            idx = (jg * LANES + lax.iota(jnp.int32, LANES)) * DIM + d
            val = plsc.load_gather(fbuf, [idx])
            obuf[so, d, pl.ds(jg * LANES, LANES)] = val
        return _

      lax.fori_loop(0, 8, dloop, 0, unroll=False)

    def step(p, sg, so, first, last):
      wait_gather(sg)
      if not first:
        wait_out(so)
      transpose(sg, so)
      issue_out(p, so)
      if not last:
        issue_gather(p + NGB, sg)

    for p in range(NGB):
      issue_gather(p, p % NGB)
    for p in range(NGB):
      step(p, p % NGB, p % NOB, first=(p < NOB), last=False)

    def body(i, _):
      base = i * NGB
      for q in range(NGB):
        step(base + q, q, (base + q) % NOB, first=False, last=False)
      return _

    lax.fori_loop(1, (NT - NGB) // NGB, body, 0, unroll=False)

    for p in range(NT - NGB, NT):
      step(p, p % NGB, p % NOB, first=False, last=True)
    for so in range(NOB):
      wait_out(so)

  return gather_kernel


_kernel = _make_kernel()


@jax.jit
def kernel(inputs, embed):
  idx_t = jnp.transpose(inputs.astype(jnp.int32))
  out_t = _kernel(idx_t, embed)
  return jnp.transpose(out_t, (2, 0, 1))


# speedup vs baseline: 1.7006x; 1.7006x over previous
import functools

import jax
import jax.numpy as jnp
from jax import lax
from jax.experimental import pallas as pl
from jax.experimental.pallas import tpu as pltpu
from jax.experimental.pallas import tpu_sc as plsc

VOCAB = 1000000
DIM = 64

NC = 2
NS = 16
NW = NC * NS

B_TOTAL = 4096 * 200
B_PER_W = B_TOTAL // NW
CHUNK = 128
NCHUNK = B_PER_W // CHUNK
NBUF = 8
LOOK = NBUF // 2


def _make_kernel():
  mesh = plsc.VectorSubcoreMesh(core_axis_name="c", subcore_axis_name="s")

  @functools.partial(
      pl.kernel,
      mesh=mesh,
      compiler_params=pltpu.CompilerParams(
          use_tc_tiling_on_sc=False, needs_layout_passes=False),
      out_type=jax.ShapeDtypeStruct((NW, NCHUNK, CHUNK, DIM), jnp.float32),
      scratch_types=[
          pltpu.VMEM((NCHUNK, CHUNK), jnp.int32),
          pltpu.VMEM((NBUF, CHUNK, DIM), jnp.float32),
          pltpu.SemaphoreType.DMA((NBUF,)),
          pltpu.SemaphoreType.DMA((NBUF,)),
      ],
  )
  def gather_kernel(idx_hbm, table_hbm, out_hbm, idx_v, bufs, gsem, ssem):
    wid = lax.axis_index("s") * NC + lax.axis_index("c")

    pltpu.sync_copy(idx_hbm.at[wid], idx_v)

    def issue_gather(b, c):
      pltpu.async_copy(table_hbm.at[idx_v.at[c]], bufs.at[b], gsem.at[b])

    def wait_gather(b):
      pltpu.make_async_copy(
          table_hbm.at[idx_v.at[0]], bufs.at[b], gsem.at[b]).wait()

    def issue_scatter(b, c):
      pltpu.async_copy(bufs.at[b], out_hbm.at[wid, c], ssem.at[b])

    def wait_scatter(b):
      pltpu.make_async_copy(
          bufs.at[b], out_hbm.at[wid, 0], ssem.at[b]).wait()


    def step(c, b):
      b2 = (b + LOOK) % NBUF
      wait_scatter(b2)
      issue_gather(b2, c + LOOK)
      wait_gather(b)
      issue_scatter(b, c)

    for c in range(LOOK):
      issue_gather(c % NBUF, c)
    for c in range(LOOK, NBUF):
      issue_gather(c % NBUF, c)
      wait_gather((c - LOOK) % NBUF)
      issue_scatter((c - LOOK) % NBUF, c - LOOK)

    def body(i, _):
      base = LOOK + (i - 1) * NBUF
      for j in range(NBUF):
        c = base + j
        step(c, (LOOK + j) % NBUF)
      return _

    n_steady = (NCHUNK - NBUF)
    assert n_steady % NBUF == 0
    lax.fori_loop(1, n_steady // NBUF + 1, body, 0, unroll=False)

    for c in range(NCHUNK - LOOK, NCHUNK):
      b = c % NBUF
      b2 = (b + LOOK) % NBUF
      wait_scatter(b2)
      wait_gather(b)
      issue_scatter(b, c)
    for c in range(NCHUNK - LOOK, NCHUNK):
      wait_scatter(c % NBUF)

  return gather_kernel


_kernel = _make_kernel()


@jax.jit
def kernel(inputs, embed):
  idx = inputs.astype(jnp.int32).reshape(NW, NCHUNK, CHUNK)
  out = _kernel(idx, embed)
  return out.reshape(inputs.shape[0], inputs.shape[1], DIM)
